# baseline (speedup 1.0000x reference)
import jax
import jax.numpy as jnp
from jax.experimental import pallas as pl
from jax.experimental.pallas import tpu as pltpu

B, H, W, C = 64, 32, 32, 768
N = B * H * W
LANES = 128
ROWS = 2048
GRID = N // ROWS
NBUF = 4


def _cam_body(x_hbm, w_ref, o_ref, buf, sem):
    i = pl.program_id(0)
    slot = jax.lax.rem(i, NBUF)

    @pl.when(i == 0)
    def _prime():
        for j in range(NBUF):
            pltpu.make_async_copy(x_hbm.at[j], buf.at[j], sem.at[j]).start()

    pltpu.make_async_copy(x_hbm.at[i], buf.at[slot], sem.at[slot]).wait()
    r = jnp.sum(buf[slot] * w_ref[...], axis=1)
    o_ref[...] = r.reshape(ROWS // LANES, LANES)

    @pl.when(i + NBUF < GRID)
    def _refill():
        pltpu.make_async_copy(x_hbm.at[i + NBUF], buf.at[slot], sem.at[slot]).start()


def kernel(conv_input, output, weight):
    x = conv_input.reshape(GRID, ROWS, C)
    w = weight.reshape(1, C)
    out = pl.pallas_call(
        _cam_body,
        grid=(GRID,),
        in_specs=[
            pl.BlockSpec(memory_space=pl.ANY),
            pl.BlockSpec((1, C), lambda i: (0, 0)),
        ],
        out_specs=pl.BlockSpec((ROWS // LANES, LANES), lambda i: (i, 0)),
        out_shape=jax.ShapeDtypeStruct((N // LANES, LANES), jnp.float32),
        scratch_shapes=[
            pltpu.VMEM((NBUF, ROWS, C), jnp.float32),
            pltpu.SemaphoreType.DMA((NBUF,)),
        ],
    )(x, w)
    return (out.reshape(B, H, W), output)


# speedup vs baseline: 1.0227x; 1.0227x over previous
import jax
import jax.numpy as jnp
from jax.experimental import pallas as pl
from jax.experimental.pallas import tpu as pltpu

B, H, W, C = 64, 32, 32, 768
N = B * H * W
LANES = 128
ROWS = 2048
GRID = N // ROWS


def _cam_body(x_ref, w_ref, o_ref):
    r = jnp.sum(x_ref[...] * w_ref[...], axis=1)
    o_ref[...] = r.reshape(ROWS // LANES, LANES)


def kernel(conv_input, output, weight):
    x = conv_input.reshape(N, C)
    w = weight.reshape(1, C)
    out = pl.pallas_call(
        _cam_body,
        grid=(GRID,),
        in_specs=[
            pl.BlockSpec((ROWS, C), lambda i: (i, 0)),
            pl.BlockSpec((1, C), lambda i: (0, 0)),
        ],
        out_specs=pl.BlockSpec((ROWS // LANES, LANES), lambda i: (i, 0)),
        out_shape=jax.ShapeDtypeStruct((N // LANES, LANES), jnp.float32),
    )(x, w)
    return (out.reshape(B, H, W), output)
